# baseline (speedup 1.0000x reference)
import functools

import jax
import jax.numpy as jnp
from jax import lax
from jax.experimental import pallas as pl
from jax.experimental.pallas import tpu as pltpu
from jax.experimental.pallas import tpu_sc as plsc

EMB = 32
B = 4096
L = 200
TOTAL = B * L
NC = 2
NS = 16
NW = NC * NS
PER_W = TOTAL // NW
CHUNK = 1280
NCHUNK = PER_W // CHUNK

VOCAB_ROWS = 1000000
PAD_COLS = 1000064
PACK_COLS = 1664

_mesh = plsc.VectorSubcoreMesh(core_axis_name="c", subcore_axis_name="s")


def _repack_body(t_ref, o_ref):
    t3 = t_ref[...].T.reshape(PACK_COLS // 4, 4, EMB)
    o_ref[...] = jnp.concatenate([t3[:, j, :] for j in range(4)], axis=1)


_repack = pl.pallas_call(
    _repack_body,
    grid=(PAD_COLS // PACK_COLS,),
    in_specs=[pl.BlockSpec((EMB, PACK_COLS), lambda i: (0, i))],
    out_specs=pl.BlockSpec((PACK_COLS // 4, 128), lambda i: (i, 0)),
    out_shape=jax.ShapeDtypeStruct((PAD_COLS // 4, 128), jnp.float32),
)


@functools.partial(
    pl.kernel,
    out_type=jax.ShapeDtypeStruct((TOTAL, 128), jnp.float32),
    mesh=_mesh,
    compiler_params=pltpu.CompilerParams(use_tc_tiling_on_sc=False),
    scratch_types=[
        pltpu.VMEM((PER_W,), jnp.int32),
        pltpu.VMEM((2, CHUNK, EMB), jnp.float32),
        pltpu.SemaphoreType.DMA,
        pltpu.SemaphoreType.DMA,
    ],
)
def _emb_lookup(x_hbm, table_hbm, out_hbm, idx_v, rows_v, gsem, wsem):
    wid = lax.axis_index("s") * NC + lax.axis_index("c")
    base = wid * PER_W
    pltpu.sync_copy(x_hbm.at[pl.ds(base, PER_W)], idx_v)

    def gather(j, slot):
        pltpu.async_copy(
            table_hbm.at[idx_v.at[pl.ds(j * CHUNK, CHUNK)]], rows_v.at[slot], gsem
        )

    def gather_wait(slot):
        pltpu.make_async_copy(
            table_hbm.at[pl.ds(0, CHUNK)], rows_v.at[slot], gsem
        ).wait()

    def write(j, slot):
        pltpu.async_copy(
            rows_v.at[slot],
            out_hbm.at[pl.ds(base + j * CHUNK, CHUNK), pl.ds(0, EMB)],
            wsem,
        )

    def write_wait(j, slot):
        pltpu.make_async_copy(
            rows_v.at[slot],
            out_hbm.at[pl.ds(base + j * CHUNK, CHUNK), pl.ds(0, EMB)],
            wsem,
        ).wait()

    gather(0, 0)

    def body(j, _):
        slot = lax.rem(j, 2)
        nslot = 1 - slot

        @pl.when(j >= 1)
        def _():
            write_wait(j - 1, nslot)

        @pl.when(j + 1 < NCHUNK)
        def _():
            gather(j + 1, nslot)

        gather_wait(slot)
        write(j, slot)
        return 0

    lax.fori_loop(0, NCHUNK, body, 0)
    write_wait(NCHUNK - 1, (NCHUNK - 1) % 2)


def kernel(x, table):
    tpadded = jnp.pad(table.T, ((0, 0), (0, PAD_COLS - VOCAB_ROWS)))
    packed = _repack(tpadded)[: VOCAB_ROWS // 4].reshape(VOCAB_ROWS, EMB)
    out = _emb_lookup(x.reshape(TOTAL).astype(jnp.int32), packed)
    return out[:, :EMB].reshape(B, L, EMB)


# speedup vs baseline: 1.3755x; 1.3755x over previous
import functools

import jax
import jax.numpy as jnp
from jax import lax
from jax.experimental import pallas as pl
from jax.experimental.pallas import tpu as pltpu
from jax.experimental.pallas import tpu_sc as plsc

EMB = 32
B = 4096
L = 200
TOTAL = B * L
NC = 2
NS = 16
NW = NC * NS
PER_W = TOTAL // NW
CHUNK = 1280
NCHUNK = PER_W // CHUNK

_mesh = plsc.VectorSubcoreMesh(core_axis_name="c", subcore_axis_name="s")


@functools.partial(
    pl.kernel,
    out_type=jax.ShapeDtypeStruct((TOTAL, 128), jnp.float32),
    mesh=_mesh,
    compiler_params=pltpu.CompilerParams(use_tc_tiling_on_sc=False),
    scratch_types=[
        pltpu.VMEM((PER_W,), jnp.int32),
        pltpu.VMEM((2, CHUNK, EMB), jnp.float32),
        pltpu.SemaphoreType.DMA,
        pltpu.SemaphoreType.DMA,
    ],
)
def _emb_lookup(x_hbm, table_hbm, out_hbm, idx_v, rows_v, gsem, wsem):
    wid = lax.axis_index("s") * NC + lax.axis_index("c")
    base = wid * PER_W
    pltpu.sync_copy(x_hbm.at[pl.ds(base, PER_W)], idx_v)

    def gather(j, slot):
        pltpu.async_copy(
            table_hbm.at[idx_v.at[pl.ds(j * CHUNK, CHUNK)]], rows_v.at[slot], gsem
        )

    def gather_wait(slot):
        pltpu.make_async_copy(
            table_hbm.at[pl.ds(0, CHUNK)], rows_v.at[slot], gsem
        ).wait()

    def write(j, slot):
        pltpu.async_copy(
            rows_v.at[slot],
            out_hbm.at[pl.ds(base + j * CHUNK, CHUNK), pl.ds(0, EMB)],
            wsem,
        )

    def write_wait(j, slot):
        pltpu.make_async_copy(
            rows_v.at[slot],
            out_hbm.at[pl.ds(base + j * CHUNK, CHUNK), pl.ds(0, EMB)],
            wsem,
        ).wait()

    gather(0, 0)

    def body(j, _):
        slot = lax.rem(j, 2)
        nslot = 1 - slot

        @pl.when(j >= 1)
        def _():
            write_wait(j - 1, nslot)

        @pl.when(j + 1 < NCHUNK)
        def _():
            gather(j + 1, nslot)

        gather_wait(slot)
        write(j, slot)
        return 0

    lax.fori_loop(0, NCHUNK, body, 0)
    write_wait(NCHUNK - 1, (NCHUNK - 1) % 2)


def kernel(x, table):
    out = _emb_lookup(x.reshape(TOTAL).astype(jnp.int32), table)
    return out[:, :EMB].reshape(B, L, EMB)
